# baseline (speedup 1.0000x reference)
import functools

import jax
import jax.numpy as jnp
from jax import lax
from jax.experimental import pallas as pl
from jax.experimental.pallas import tpu as pltpu
from jax.experimental.pallas import tpu_sc as plsc

B = 16384
NC = 2
NS = 16
NW = NC * NS
BPW = B // NW
CH = 128
NCH = BPW // CH


L = 16
PERIOD = 80


def _body(idx_hbm, betas_hbm, go_hbm, tr_hbm, bp_hbm,
          betas_out, go_out, tr_out, bp_out,
          idx_v, go_v, tr_v, bp_v, beta_row, beta_v, gsem):
    c = lax.axis_index("c")
    s = lax.axis_index("s")
    wid = s * NC + c
    base = wid * BPW

    pltpu.sync_copy(idx_hbm.at[pl.ds(wid * NCH, NCH)], idx_v)

    copies = []
    for j in range(NCH):
        row = idx_v.at[j]
        dst = pl.ds(j * CH, CH)
        copies.append(pltpu.async_copy(go_hbm.at[row], go_v.at[dst], gsem))
        copies.append(pltpu.async_copy(tr_hbm.at[row], tr_v.at[dst], gsem))
        copies.append(pltpu.async_copy(bp_hbm.at[row], bp_v.at[dst], gsem))

    pltpu.sync_copy(betas_hbm, beta_row.at[pl.ds(0, 10)])
    lane = lax.iota(jnp.int32, 16)
    row_v = beta_row[...]
    dnums = lax.GatherDimensionNumbers(
        offset_dims=(), collapsed_slice_dims=(0,), start_index_map=(0,))
    vregs = [
        lax.gather(row_v,
                   lax.rem(lane + (k * L) % 10, 10)[:, None],
                   dnums, slice_sizes=(1,),
                   mode=lax.GatherScatterMode.PROMISE_IN_BOUNDS)
        for k in range(PERIOD // L)
    ]

    def fill(i, carry):
        off = i * PERIOD
        for k in range(PERIOD // L):
            beta_v[pl.ds(off + k * L, L)] = vregs[k]
        return carry

    lax.fori_loop(0, BPW * 10 // PERIOD, fill, 0)
    pltpu.sync_copy(beta_v, betas_out.at[pl.ds(base * 10, BPW * 10)])

    for cp in copies:
        cp.wait()
    pltpu.sync_copy(go_v, go_out.at[pl.ds(base, BPW)])
    pltpu.sync_copy(tr_v, tr_out.at[pl.ds(base, BPW)])
    pltpu.sync_copy(bp_v, bp_out.at[pl.ds(base, BPW)])


@jax.jit
def _run(idx2, betas_w, go_w, tr_w, bp_w):
    mesh = plsc.VectorSubcoreMesh(core_axis_name="c", subcore_axis_name="s")
    f = pl.kernel(
        _body,
        mesh=mesh,
        compiler_params=pltpu.CompilerParams(use_tc_tiling_on_sc=False),
        out_type=(
            jax.ShapeDtypeStruct((B * 10,), jnp.float32),
            jax.ShapeDtypeStruct((B, 3), jnp.float32),
            jax.ShapeDtypeStruct((B, 3), jnp.float32),
            jax.ShapeDtypeStruct((B, 69), jnp.float32),
        ),
        scratch_types=[
            pltpu.VMEM((NCH, CH), jnp.int32),
            pltpu.VMEM((BPW, 3), jnp.float32),
            pltpu.VMEM((BPW, 3), jnp.float32),
            pltpu.VMEM((BPW, 69), jnp.float32),
            pltpu.VMEM((L,), jnp.float32),
            pltpu.VMEM((BPW * 10,), jnp.float32),
            pltpu.SemaphoreType.DMA,
        ],
    )
    return f(idx2, betas_w, go_w, tr_w, bp_w)


def kernel(frame_ids, betas_w, global_orient_w, transl_w, body_pose_w):
    idx2 = frame_ids.astype(jnp.int32).reshape(NW * NCH, CH)
    betas_flat, go, tr, bp = _run(
        idx2, betas_w.reshape(10), global_orient_w, transl_w, body_pose_w)
    fi = frame_ids
    go = jnp.take(global_orient_w, fi, axis=0)
    tr = jnp.take(transl_w, fi, axis=0)
    return (betas_flat.reshape(B, 10), go, tr, bp)


# speedup vs baseline: 1.2507x; 1.2507x over previous
import functools

import jax
import jax.numpy as jnp
from jax import lax
from jax.experimental import pallas as pl
from jax.experimental.pallas import tpu as pltpu
from jax.experimental.pallas import tpu_sc as plsc

NUM_FRAMES = 100000
B = 16384
NC = 2
NS = 16
NW = NC * NS
BPW = B // NW
CH = 128
NCH = BPW // CH
L = 16
PERIOD = 80

DN = 3
DB = 69
G3 = NUM_FRAMES * DN // L
G69 = NUM_FRAMES * DB // L
MB = 6

NV = BPW * DN
BV = BPW * DB


def _dg(v, idx):
    dn = lax.GatherDimensionNumbers(
        offset_dims=(), collapsed_slice_dims=(0,), start_index_map=(0,))
    return lax.gather(v, idx[:, None], dn, slice_sizes=(1,),
                      mode=lax.GatherScatterMode.PROMISE_IN_BOUNDS)


def _body(idx_hbm, betas_hbm, go_hbm, tr_hbm, bp_hbm,
          betas_out, go_out, tr_out, bp_out,
          idx_v, ga, gb, bi,
          w3a0, w3b0, wta0, wtb0, wbp0,
          w3a1, w3b1, wta1, wtb1, wbp1,
          go_v, tr_v, bp_v, beta_row, beta_v,
          sem0, sem1):
    c = lax.axis_index("c")
    s = lax.axis_index("s")
    wid = s * NC + c
    lane = lax.iota(jnp.int32, L)

    pltpu.sync_copy(idx_hbm.at[pl.ds(wid * NCH, NCH)], idx_v)

    for j in range(NCH):
        for v in range(CH // L):
            sl = pl.ds(v * L, L)
            iv = idx_v[j, sl]
            g3 = (iv * DN) >> 4
            ga[j, sl] = g3
            gb[j, sl] = jnp.minimum(g3 + 1, G3 - 1)
            g69 = (iv * DB) >> 4
            for m in range(MB):
                bi[j * MB + m, sl] = jnp.minimum(g69 + m, G69 - 1)

    nbuf = ((w3a0, w3b0, wta0, wtb0, wbp0),
            (w3a1, w3b1, wta1, wtb1, wbp1))
    sems = (sem0, sem1)

    def fire(j):
        w3a, w3b, wta, wtb, wbp = nbuf[j & 1]
        sm = sems[j & 1]
        cps = [
            pltpu.async_copy(go_hbm.at[ga.at[j]], w3a, sm),
            pltpu.async_copy(go_hbm.at[gb.at[j]], w3b, sm),
            pltpu.async_copy(tr_hbm.at[ga.at[j]], wta, sm),
            pltpu.async_copy(tr_hbm.at[gb.at[j]], wtb, sm),
        ]
        for m in range(MB):
            cps.append(pltpu.async_copy(
                bp_hbm.at[bi.at[j * MB + m]],
                wbp.at[pl.ds(m * CH, CH)], sm))
        return cps

    inflight = fire(0)

    pltpu.sync_copy(betas_hbm, beta_row.at[pl.ds(0, 10)])
    row_v = beta_row[...]
    vregs = [_dg(row_v, lax.rem(lane + (k * L) % 10, 10))
             for k in range(PERIOD // L)]

    def bfill(i, carry):
        off = i * PERIOD
        for k in range(PERIOD // L):
            beta_v[pl.ds(off + k * L, L)] = vregs[k]
        return carry

    lax.fori_loop(0, BPW * 10 // PERIOD, bfill, 0)
    pltpu.sync_copy(beta_v, betas_out.at[pl.ds(wid * BPW * 10, BPW * 10)])

    for j in range(NCH):
        nxt = fire(j + 1) if j + 1 < NCH else None
        for cp in inflight:
            cp.wait()
        inflight = nxt
        w3a, w3b, wta, wtb, wbp = nbuf[j & 1]

        def ext(t, carry):
            grp = idx_v[j, pl.ds((t >> 4) << 4, L)]
            ib = _dg(grp, jnp.broadcast_to(t & (L - 1), (L,)))
            k3 = DN * (j * CH) + DN * t
            k69 = DB * (j * CH) + DB * t

            r3 = (DN * ib) & (L - 1)
            la3 = lane + r3
            rix3 = la3 & (L - 1)
            sel3 = la3 < L
            va = _dg(w3a[t], rix3)
            vb = _dg(w3b[t], rix3)
            go_v[pl.ds(k3, L)] = jnp.where(sel3, va, vb)
            vc = _dg(wta[t], rix3)
            vd = _dg(wtb[t], rix3)
            tr_v[pl.ds(k3, L)] = jnp.where(sel3, vc, vd)

            r69 = (DB * ib) & (L - 1)
            la69 = lane + r69
            rix69 = la69 & (L - 1)
            sel69 = la69 < L
            rots = [_dg(wbp[m * CH + t], rix69) for m in range(MB)]
            for tt in range(MB - 1):
                bp_v[pl.ds(k69 + tt * L, L)] = jnp.where(
                    sel69, rots[tt], rots[tt + 1])
            return carry

        lax.fori_loop(0, CH, ext, 0)

    pltpu.sync_copy(go_v.at[pl.ds(0, NV)], go_out.at[pl.ds(wid * NV, NV)])
    pltpu.sync_copy(tr_v.at[pl.ds(0, NV)], tr_out.at[pl.ds(wid * NV, NV)])
    pltpu.sync_copy(bp_v.at[pl.ds(0, BV)], bp_out.at[pl.ds(wid * BV, BV)])


@jax.jit
def _run(idx2, betas_w, go16, tr16, bp16):
    mesh = plsc.VectorSubcoreMesh(core_axis_name="c", subcore_axis_name="s")
    win_i32 = pltpu.VMEM((CH, L), jnp.float32)
    f = pl.kernel(
        _body,
        mesh=mesh,
        compiler_params=pltpu.CompilerParams(use_tc_tiling_on_sc=False),
        out_type=(
            jax.ShapeDtypeStruct((B * 10,), jnp.float32),
            jax.ShapeDtypeStruct((B * DN,), jnp.float32),
            jax.ShapeDtypeStruct((B * DN,), jnp.float32),
            jax.ShapeDtypeStruct((B * DB,), jnp.float32),
        ),
        scratch_types=[
            pltpu.VMEM((NCH, CH), jnp.int32),
            pltpu.VMEM((NCH, CH), jnp.int32),
            pltpu.VMEM((NCH, CH), jnp.int32),
            pltpu.VMEM((NCH * MB, CH), jnp.int32),
            win_i32, win_i32, win_i32, win_i32,
            pltpu.VMEM((MB * CH, L), jnp.float32),
            win_i32, win_i32, win_i32, win_i32,
            pltpu.VMEM((MB * CH, L), jnp.float32),
            pltpu.VMEM((NV + L,), jnp.float32),
            pltpu.VMEM((NV + L,), jnp.float32),
            pltpu.VMEM((BV + L,), jnp.float32),
            pltpu.VMEM((L,), jnp.float32),
            pltpu.VMEM((BPW * 10,), jnp.float32),
            pltpu.SemaphoreType.DMA,
            pltpu.SemaphoreType.DMA,
        ],
    )
    return f(idx2, betas_w, go16, tr16, bp16)


def kernel(frame_ids, betas_w, global_orient_w, transl_w, body_pose_w):
    idx2 = frame_ids.astype(jnp.int32).reshape(NW * NCH, CH)
    betas_f, go_f, tr_f, bp_f = _run(
        idx2,
        betas_w.reshape(10),
        global_orient_w.reshape(G3, L),
        transl_w.reshape(G3, L),
        body_pose_w.reshape(G69, L),
    )
    return (betas_f.reshape(B, 10), go_f.reshape(B, DN),
            tr_f.reshape(B, DN), bp_f.reshape(B, DB))


# speedup vs baseline: 2.2217x; 1.7764x over previous
import functools

import jax
import jax.numpy as jnp
from jax import lax
from jax.experimental import pallas as pl
from jax.experimental.pallas import tpu as pltpu
from jax.experimental.pallas import tpu_sc as plsc

NUM_FRAMES = 100000
B = 16384
NC = 2
NS = 16
NW = NC * NS
BPW = B // NW
CH = 128
NCH = BPW // CH
L = 16
PERIOD = 80

DN = 3
DB = 69
FAT = 128
NV = BPW * DN
BV = BPW * DB

FBLK = 2048


def _dg(v, idx):
    dn = lax.GatherDimensionNumbers(
        offset_dims=(), collapsed_slice_dims=(0,), start_index_map=(0,))
    return lax.gather(v, idx[:, None], dn, slice_sizes=(1,),
                      mode=lax.GatherScatterMode.PROMISE_IN_BOUNDS)


def _pack_body(bp_ref, go_ref, tr_ref, out_ref):
    out_ref[:, 0:DB] = bp_ref[...].T
    out_ref[:, DB:DB + DN] = go_ref[...].T
    out_ref[:, DB + DN:DB + 2 * DN] = tr_ref[...].T


def _pack(bp_t, go_t, tr_t):
    grid = (NUM_FRAMES + FBLK - 1) // FBLK
    return pl.pallas_call(
        _pack_body,
        grid=(grid,),
        in_specs=[
            pl.BlockSpec((DB, FBLK), lambda n: (0, n)),
            pl.BlockSpec((DN, FBLK), lambda n: (0, n)),
            pl.BlockSpec((DN, FBLK), lambda n: (0, n)),
        ],
        out_specs=pl.BlockSpec((FBLK, FAT), lambda n: (n, 0)),
        out_shape=jax.ShapeDtypeStruct((NUM_FRAMES, FAT), jnp.float32),
    )(bp_t, go_t, tr_t)


def _body(idx_hbm, betas_hbm, fat_hbm,
          betas_out, go_out, tr_out, bp_out,
          idx_v, win0, win1, go_v, tr_v, bp_v, beta_row, beta_v,
          sem0, sem1):
    c = lax.axis_index("c")
    s = lax.axis_index("s")
    wid = s * NC + c
    lane = lax.iota(jnp.int32, L)

    pltpu.sync_copy(idx_hbm.at[pl.ds(wid * NCH, NCH)], idx_v)

    wins = (win0, win1)
    sems = (sem0, sem1)

    def fire(j):
        return pltpu.async_copy(fat_hbm.at[idx_v.at[j]], wins[j & 1],
                                sems[j & 1])

    inflight = fire(0)

    pltpu.sync_copy(betas_hbm, beta_row.at[pl.ds(0, 10)])
    row_v = beta_row[...]
    vregs = [_dg(row_v, lax.rem(lane + (k * L) % 10, 10))
             for k in range(PERIOD // L)]

    def bfill(i, carry):
        off = i * PERIOD
        for k in range(PERIOD // L):
            beta_v[pl.ds(off + k * L, L)] = vregs[k]
        return carry

    lax.fori_loop(0, BPW * 10 // PERIOD, bfill, 0)
    pltpu.sync_copy(beta_v, betas_out.at[pl.ds(wid * BPW * 10, BPW * 10)])

    rix_go = (lane + DB - 4 * L) & (L - 1)
    rix_tr = (lane + DB + DN - 4 * L) & (L - 1)

    for j in range(NCH):
        nxt = fire(j + 1) if j + 1 < NCH else None
        inflight.wait()
        inflight = nxt
        win = wins[j & 1]

        def ext(t, carry):
            k3 = DN * (j * CH) + DN * t
            k69 = DB * (j * CH) + DB * t
            w4 = win[t, pl.ds(4 * L, L)]
            for m in range(4):
                bp_v[pl.ds(k69 + m * L, L)] = win[t, pl.ds(m * L, L)]
            bp_v[pl.ds(k69 + 4 * L, L)] = w4
            go_v[pl.ds(k3, L)] = _dg(w4, rix_go)
            tr_v[pl.ds(k3, L)] = _dg(w4, rix_tr)
            return carry

        lax.fori_loop(0, CH, ext, 0)

    pltpu.sync_copy(go_v.at[pl.ds(0, NV)], go_out.at[pl.ds(wid * NV, NV)])
    pltpu.sync_copy(tr_v.at[pl.ds(0, NV)], tr_out.at[pl.ds(wid * NV, NV)])
    pltpu.sync_copy(bp_v.at[pl.ds(0, BV)], bp_out.at[pl.ds(wid * BV, BV)])


@jax.jit
def _run(idx2, betas_w, fat):
    mesh = plsc.VectorSubcoreMesh(core_axis_name="c", subcore_axis_name="s")
    f = pl.kernel(
        _body,
        mesh=mesh,
        compiler_params=pltpu.CompilerParams(use_tc_tiling_on_sc=False),
        out_type=(
            jax.ShapeDtypeStruct((B * 10,), jnp.float32),
            jax.ShapeDtypeStruct((B * DN,), jnp.float32),
            jax.ShapeDtypeStruct((B * DN,), jnp.float32),
            jax.ShapeDtypeStruct((B * DB,), jnp.float32),
        ),
        scratch_types=[
            pltpu.VMEM((NCH, CH), jnp.int32),
            pltpu.VMEM((CH, FAT), jnp.float32),
            pltpu.VMEM((CH, FAT), jnp.float32),
            pltpu.VMEM((NV + L,), jnp.float32),
            pltpu.VMEM((NV + L,), jnp.float32),
            pltpu.VMEM((BV + L,), jnp.float32),
            pltpu.VMEM((L,), jnp.float32),
            pltpu.VMEM((BPW * 10,), jnp.float32),
            pltpu.SemaphoreType.DMA,
            pltpu.SemaphoreType.DMA,
        ],
    )
    return f(idx2, betas_w, fat)


def kernel(frame_ids, betas_w, global_orient_w, transl_w, body_pose_w):
    idx2 = frame_ids.astype(jnp.int32).reshape(NW * NCH, CH)
    fat = _pack(body_pose_w.T, global_orient_w.T, transl_w.T)
    betas_f, go_f, tr_f, bp_f = _run(idx2, betas_w.reshape(10), fat)
    return (betas_f.reshape(B, 10), go_f.reshape(B, DN),
            tr_f.reshape(B, DN), bp_f.reshape(B, DB))
